# baseline (speedup 1.0000x reference)
import functools

import numpy as np
import jax
import jax.numpy as jnp
from jax import lax
from jax.experimental import pallas as pl
from jax.experimental.pallas import tpu as pltpu
from jax.experimental.pallas import tpu_sc as plsc

_DIAG = 2
_SEQ = 512
_FLAT = _SEQ * _SEQ
_NSLAB = 128
_NW = 32
_SG = 8
_LANES = 16
_WINMAX = 12288
_COLMAX = 2048

_r, _c = np.triu_indices(_SEQ, k=_DIAG)
_N = _r.size
_idx64 = (_r.astype(np.int64) * _SEQ + _c.astype(np.int64))
_NCOV = _N - 1


def _build_chunks():
    chunks = []
    col = 0
    while col < _NCOV:
        wb = int(_idx64[col] // 128) * 128
        cl = 128
        while cl < _COLMAX and col + cl < _NCOV:
            nl = cl + 128
            end = min(col + nl, _NCOV) - 1
            wl = -(-(int(_idx64[end]) + 1 - wb) // 128) * 128
            if wl > _WINMAX:
                break
            cl = nl
        cl = min(cl, _NCOV - col)
        end = col + cl - 1
        wl = -(-(int(_idx64[end]) + 1 - wb) // 128) * 128
        chunks.append((col, cl, wb, wl))
        col += cl
    return chunks

_CHUNKS = _build_chunks()

_qsegs = []
_sub_local = [0] * _NW
_qparts = []
_goff = 0
for _k, (_cb, _cl, _wb, _wl) in enumerate(_CHUNKS):
    _w = _k % _NW
    _seg = (_idx64[_cb:_cb + _cl] - _wb).astype(np.int32)
    _pad = -(-_cl // 128) * 128
    _segp = np.zeros(_pad, np.int32)
    _segp[:_cl] = _seg
    _qsegs.append((_k, _w, _sub_local[_w], _goff))
    _sub_local[_w] += _pad
    _qparts.append(_segp)
    _goff += _pad
_QTOT = _goff
_QLOCMAX = max(_sub_local)
_q_host = np.concatenate(_qparts)


def _tri_gather_sc(in2, qtab):
    mesh = plsc.VectorSubcoreMesh(core_axis_name="c", subcore_axis_name="s")

    @functools.partial(
        pl.kernel,
        out_type=jax.ShapeDtypeStruct((_NSLAB, _N), jnp.float32),
        mesh=mesh,
        scratch_types=[
            pltpu.VMEM((_QLOCMAX,), jnp.int32),
            pltpu.VMEM((_SG, _WINMAX), jnp.float32),
            pltpu.VMEM((_SG, _COLMAX), jnp.float32),
            pltpu.SemaphoreType.DMA,
        ],
        compiler_params=pltpu.CompilerParams(needs_layout_passes=False),
    )
    def body(in_hbm, q_hbm, out_hbm, q_v, win_v, outb_v, sem):
        wid = lax.axis_index("s") * 2 + lax.axis_index("c")

        for k, w, loff, goff in _qsegs:
            cb, cl, wb, wl = _CHUNKS[k]
            pad = -(-cl // 128) * 128

            @pl.when(wid == w)
            def _(loff=loff, goff=goff, pad=pad):
                pltpu.sync_copy(
                    q_hbm.at[pl.ds(goff, pad)], q_v.at[pl.ds(loff, pad)]
                )

        def group_body(g, carry):
            row0 = pl.multiple_of(g * _SG, _SG)
            for k, w, loff, goff in _qsegs:
                cb, cl, wb, wl = _CHUNKS[k]

                @pl.when(wid == w)
                def _(cb=cb, cl=cl, wb=wb, wl=wl, loff=loff):
                    pltpu.sync_copy(
                        in_hbm.at[pl.ds(row0, _SG), pl.ds(wb, wl)],
                        win_v.at[:, pl.ds(0, wl)],
                    )

                    def jbody(j, c2):
                        jvec = jnp.full((_LANES,), 0, jnp.int32) + j

                        def tbody(t, c3):
                            q16 = q_v[pl.ds(loff + t * _LANES, _LANES)]
                            vals = plsc.load_gather(win_v, [jvec, q16])
                            outb_v[j, pl.ds(t * _LANES, _LANES)] = vals
                            return c3

                        lax.fori_loop(0, cl // _LANES, tbody, 0)
                        return c2

                    lax.fori_loop(0, _SG, jbody, 0)
                    pltpu.sync_copy(
                        outb_v.at[:, pl.ds(0, cl)],
                        out_hbm.at[pl.ds(row0, _SG), pl.ds(cb, cl)],
                    )

            return carry

        lax.fori_loop(0, _NSLAB // _SG, group_body, 0)

    return body(in2, qtab)


def kernel(inputs):
    b, ch, s, _ = inputs.shape
    in2 = inputs.reshape(_NSLAB, _FLAT)
    out = _tri_gather_sc(in2, jnp.asarray(_q_host))
    out = out.reshape(b, ch, _N)
    return out.at[:, :, _N - 1].set(inputs[:, :, _SEQ - _DIAG - 1, _SEQ - 1])


# speedup vs baseline: 2.1431x; 2.1431x over previous
import functools

import numpy as np
import jax
import jax.numpy as jnp
from jax import lax
from jax.experimental import pallas as pl
from jax.experimental.pallas import tpu as pltpu
from jax.experimental.pallas import tpu_sc as plsc

_DIAG = 2
_SEQ = 512
_FLAT = _SEQ * _SEQ
_NSLAB = 128
_NW = 32
_SG = 8
_LANES = 16
_WINMAX = 6144
_COLMAX = 1536
_COLCAP = 4608

_r, _c = np.triu_indices(_SEQ, k=_DIAG)
_N = _r.size
_idx64 = (_r.astype(np.int64) * _SEQ + _c.astype(np.int64))
_NCOV = _N - 1


def _build_chunks():
    chunks = []
    col = 0
    while col < _NCOV:
        wb = int(_idx64[col] // 128) * 128
        cl = 128
        while cl < _COLMAX and col + cl < _NCOV:
            nl = cl + 128
            end = min(col + nl, _NCOV) - 1
            wl = -(-(int(_idx64[end]) + 1 - wb) // 128) * 128
            if wl > _WINMAX:
                break
            cl = nl
        cl = min(cl, _NCOV - col)
        end = col + cl - 1
        wl = -(-(int(_idx64[end]) + 1 - wb) // 128) * 128
        chunks.append((col, cl, wb, wl))
        col += cl
    return chunks

_CHUNKS = _build_chunks()

_BINS = [[0, 0, []] for _ in range(_NW)]
for _k in sorted(range(len(_CHUNKS)), key=lambda k: -_CHUNKS[k][3]):
    _cl = _CHUNKS[_k][1]
    _cand = [j for j in range(_NW) if _BINS[j][1] + _cl <= _COLCAP]
    _i = min(_cand, key=lambda j: _BINS[j][0])
    _BINS[_i][0] += _CHUNKS[_k][3]
    _BINS[_i][1] += _cl
    _BINS[_i][2].append(_k)
_WINCAP = max(b[0] for b in _BINS)
_COLTOT = max(b[1] for b in _BINS)

_PLAN = []
_qparts = []
_goff = 0
for _w in range(_NW):
    _items = []
    _winoff = 0
    _oboff = 0
    for _k in _BINS[_w][2]:
        _cb, _cl, _wb, _wl = _CHUNKS[_k]
        _qparts.append((_idx64[_cb:_cb + _cl] - _wb + _winoff).astype(np.int32))
        _items.append((_cb, _cl, _wb, _wl, _winoff, _oboff))
        _winoff += _wl
        _oboff += _cl
    _qlen = _oboff
    _PLAN.append((_goff, _qlen, _items))
    _goff += _qlen
_q_host = np.concatenate(_qparts)


def _tri_gather_sc(in2, qtab):
    mesh = plsc.VectorSubcoreMesh(core_axis_name="c", subcore_axis_name="s")

    @functools.partial(
        pl.kernel,
        out_type=jax.ShapeDtypeStruct((_NSLAB, _N), jnp.float32),
        mesh=mesh,
        scratch_types=[
            pltpu.VMEM((_COLCAP,), jnp.int32),
            pltpu.VMEM((_SG, _WINCAP), jnp.float32),
            pltpu.VMEM((_SG, _COLCAP), jnp.float32),
            pltpu.SemaphoreType.DMA,
            pltpu.SemaphoreType.DMA,
        ],
        compiler_params=pltpu.CompilerParams(needs_layout_passes=False),
    )
    def body(in_hbm, q_hbm, out_hbm, q_v, win_v, outb_v, sst, swr):
        wid = lax.axis_index("s") * 2 + lax.axis_index("c")
        jvecs = [jnp.full((_LANES,), j, jnp.int32) for j in range(_SG)]

        for w in range(_NW):
            goff, qlen, items = _PLAN[w]

            @pl.when(wid == w)
            def _(goff=goff, qlen=qlen):
                pltpu.sync_copy(
                    q_hbm.at[pl.ds(goff, qlen)], q_v.at[pl.ds(0, qlen)]
                )

        def group_body(g, carry):
            row0 = pl.multiple_of(g * _SG, _SG)
            prev0 = pl.multiple_of(0, _SG)
            for w in range(_NW):
                goff, qlen, items = _PLAN[w]

                @pl.when(wid == w)
                def _(qlen=qlen, items=items):
                    stages = []
                    for cb, cl, wb, wl, winoff, oboff in items:
                        stages.append(pltpu.async_copy(
                            in_hbm.at[pl.ds(row0, _SG), pl.ds(wb, wl)],
                            win_v.at[:, pl.ds(winoff, wl)],
                            sst,
                        ))

                    @pl.when(g > 0)
                    def _():
                        for cb, cl, wb, wl, winoff, oboff in items:
                            pltpu.make_async_copy(
                                outb_v.at[:, pl.ds(oboff, cl)],
                                out_hbm.at[pl.ds(prev0, _SG), pl.ds(cb, cl)],
                                swr,
                            ).wait()

                    for d in stages:
                        d.wait()

                    def tbody(t, c3):
                        base = t * (2 * _LANES)
                        qa = q_v[pl.ds(base, _LANES)]
                        qb = q_v[pl.ds(base + _LANES, _LANES)]
                        for j in range(_SG):
                            va = plsc.load_gather(win_v, [jvecs[j], qa])
                            vb = plsc.load_gather(win_v, [jvecs[j], qb])
                            outb_v[j, pl.ds(base, _LANES)] = va
                            outb_v[j, pl.ds(base + _LANES, _LANES)] = vb
                        return c3

                    lax.fori_loop(0, qlen // (2 * _LANES), tbody, 0)

                    for cb, cl, wb, wl, winoff, oboff in items:
                        pltpu.async_copy(
                            outb_v.at[:, pl.ds(oboff, cl)],
                            out_hbm.at[pl.ds(row0, _SG), pl.ds(cb, cl)],
                            swr,
                        )

            return carry

        lax.fori_loop(0, _NSLAB // _SG, group_body, 0)

        last0 = pl.multiple_of(_NSLAB - _SG, _SG)
        for w in range(_NW):
            goff, qlen, items = _PLAN[w]

            @pl.when(wid == w)
            def _(items=items):
                for cb, cl, wb, wl, winoff, oboff in items:
                    pltpu.make_async_copy(
                        outb_v.at[:, pl.ds(oboff, cl)],
                        out_hbm.at[pl.ds(last0, _SG), pl.ds(cb, cl)],
                        swr,
                    ).wait()

    return body(in2, qtab)


def kernel(inputs):
    b, ch, s, _ = inputs.shape
    in2 = inputs.reshape(_NSLAB, _FLAT)
    out = _tri_gather_sc(in2, jnp.asarray(_q_host))
    out = out.reshape(b, ch, _N)
    return out.at[:, :, _N - 1].set(inputs[:, :, _SEQ - _DIAG - 1, _SEQ - 1])
